# baseline (speedup 1.0000x reference)
import functools

import jax
import jax.numpy as jnp
from jax import lax
from jax.experimental import pallas as pl
from jax.experimental.pallas import tpu as pltpu
from jax.experimental.pallas import tpu_sc as plsc

_NC = 2
_NS = 16
_NW = _NC * _NS
_CHUNK = 128
_GRP = 1


def _sc_mesh():
    return plsc.VectorSubcoreMesh(core_axis_name="c", subcore_axis_name="s")


def _deg_partials(dst, ones8, zeros8, M, n_chunks):
    rpt = M // _NS

    def body(dst_hbm, ones_hbm, zeros_hbm, out_hbm, ones_v, zbuf_v, dst_v, acc_sh):
        c = lax.axis_index("c")
        s = lax.axis_index("s")
        wid = c * _NS + s
        pltpu.sync_copy(zeros_hbm, zbuf_v)
        for k in range(rpt // _CHUNK):
            pltpu.sync_copy(zbuf_v, acc_sh.at[pl.ds(s * rpt + k * _CHUNK, _CHUNK)])
        pltpu.sync_copy(dst_hbm.at[wid], dst_v)
        pltpu.sync_copy(ones_hbm, ones_v)
        plsc.subcore_barrier()

        def step(i, carry):
            pltpu.sync_copy(ones_v, acc_sh.at[dst_v.at[i]], add=True)
            return carry

        lax.fori_loop(0, n_chunks, step, 0)
        plsc.subcore_barrier()
        pltpu.sync_copy(acc_sh.at[pl.ds(s * rpt, rpt)],
                        out_hbm.at[c, pl.ds(s * rpt, rpt)])

    fn = pl.kernel(
        body,
        out_type=jax.ShapeDtypeStruct((_NC, M, 8), jnp.float32),
        mesh=_sc_mesh(),
        scratch_types=[
            pltpu.VMEM((_GRP * _CHUNK, 8), jnp.float32),
            pltpu.VMEM((_CHUNK, 8), jnp.float32),
            pltpu.VMEM((n_chunks, _GRP * _CHUNK), jnp.int32),
            pltpu.VMEM_SHARED((M, 8), jnp.float32),
        ],
    )
    return fn(dst, ones8, zeros8)


def _agg_partials(g, src, dst, zeros_h, M, n_chunks, H):
    rpt = M // _NS

    def body(g_hbm, src_hbm, dst_hbm, zeros_hbm, out_hbm,
             src_v, dst_v, rows0_v, rows1_v, zbuf_v, acc_sh, sem0):
        c = lax.axis_index("c")
        s = lax.axis_index("s")
        wid = c * _NS + s
        pltpu.sync_copy(zeros_hbm, zbuf_v)
        for k in range(rpt // _CHUNK):
            pltpu.sync_copy(zbuf_v, acc_sh.at[pl.ds(s * rpt + k * _CHUNK, _CHUNK)])
        pltpu.sync_copy(src_hbm.at[wid], src_v)
        pltpu.sync_copy(dst_hbm.at[wid], dst_v)
        plsc.subcore_barrier()

        bufs = (rows0_v, rows1_v)
        pltpu.async_copy(g_hbm.at[src_v.at[0]], rows0_v, sem0).wait()

        def step(jj, carry):
            for b in range(2):
                j = 2 * jj + b
                nxt = jnp.minimum(j + 1, n_chunks - 1)
                d = pltpu.async_copy(g_hbm.at[src_v.at[nxt]], bufs[1 - b], sem0)
                pltpu.sync_copy(bufs[b], acc_sh.at[dst_v.at[j]], add=True)
                d.wait()
            return carry

        lax.fori_loop(0, n_chunks // 2, step, 0)
        plsc.subcore_barrier()
        pltpu.sync_copy(acc_sh.at[pl.ds(s * rpt, rpt)],
                        out_hbm.at[c, pl.ds(s * rpt, rpt)])

    fn = pl.kernel(
        body,
        out_type=jax.ShapeDtypeStruct((_NC, M, H), jnp.float32),
        mesh=_sc_mesh(),
        scratch_types=[
            pltpu.VMEM((n_chunks, _GRP * _CHUNK), jnp.int32),
            pltpu.VMEM((n_chunks, _GRP * _CHUNK), jnp.int32),
            pltpu.VMEM((_GRP * _CHUNK, H), jnp.float32),
            pltpu.VMEM((_GRP * _CHUNK, H), jnp.float32),
            pltpu.VMEM((_CHUNK, H), jnp.float32),
            pltpu.VMEM_SHARED((M, H), jnp.float32),
            pltpu.SemaphoreType.DMA,
        ],
        compiler_params=pltpu.CompilerParams(use_tc_tiling_on_sc=False),
    )
    return fn(g, src, dst, zeros_h)


def _dinv(dp_ref):
    return lax.rsqrt(dp_ref[0, :, 0:1] + dp_ref[1, :, 0:1] + 1.0)


def _tc_first(xp, W1, degp, R):
    M, F = xp.shape
    H = W1.shape[1]

    def body(x_ref, w_ref, dp_ref, g_ref):
        h = jnp.dot(x_ref[...], w_ref[...], preferred_element_type=jnp.float32)
        g_ref[...] = _dinv(dp_ref) * h

    return pl.pallas_call(
        body,
        grid=(M // R,),
        in_specs=[
            pl.BlockSpec((R, F), lambda i: (i, 0)),
            pl.BlockSpec((F, H), lambda i: (0, 0)),
            pl.BlockSpec((2, R, 8), lambda i: (0, i, 0)),
        ],
        out_specs=pl.BlockSpec((R, H), lambda i: (i, 0)),
        out_shape=jax.ShapeDtypeStruct((M, H), jnp.float32),
    )(xp, W1, degp)


def _tc_mid(acc, g, degp, b, W, R):
    M, H = g.shape
    H2 = W.shape[1]

    def body(a_ref, g_ref, dp_ref, b_ref, w_ref, o_ref):
        d = _dinv(dp_ref)
        t = jnp.maximum(d * (a_ref[0] + a_ref[1] + g_ref[...]) + b_ref[...], 0.0)
        o_ref[...] = d * jnp.dot(t, w_ref[...], preferred_element_type=jnp.float32)

    return pl.pallas_call(
        body,
        grid=(M // R,),
        in_specs=[
            pl.BlockSpec((2, R, H), lambda i: (0, i, 0)),
            pl.BlockSpec((R, H), lambda i: (i, 0)),
            pl.BlockSpec((2, R, 8), lambda i: (0, i, 0)),
            pl.BlockSpec((1, H), lambda i: (0, 0)),
            pl.BlockSpec((H, H2), lambda i: (0, 0)),
        ],
        out_specs=pl.BlockSpec((R, H2), lambda i: (i, 0)),
        out_shape=jax.ShapeDtypeStruct((M, H2), jnp.float32),
    )(acc, g, degp, b, W)


def _tc_head(acc, g, degp, b, Wh, bh, R):
    M, H = g.shape
    A = Wh.shape[1]

    def body(a_ref, g_ref, dp_ref, b_ref, w_ref, bh_ref, o_ref):
        d = _dinv(dp_ref)
        t = jnp.maximum(d * (a_ref[0] + a_ref[1] + g_ref[...]) + b_ref[...], 0.0)
        o_ref[...] = jnp.dot(t, w_ref[...], preferred_element_type=jnp.float32) + bh_ref[...]

    return pl.pallas_call(
        body,
        grid=(M // R,),
        in_specs=[
            pl.BlockSpec((2, R, H), lambda i: (0, i, 0)),
            pl.BlockSpec((R, H), lambda i: (i, 0)),
            pl.BlockSpec((2, R, 8), lambda i: (0, i, 0)),
            pl.BlockSpec((1, H), lambda i: (0, 0)),
            pl.BlockSpec((H, A), lambda i: (0, 0)),
            pl.BlockSpec((1, A), lambda i: (0, 0)),
        ],
        out_specs=pl.BlockSpec((R, A), lambda i: (i, 0)),
        out_shape=jax.ShapeDtypeStruct((M, A), jnp.float32),
    )(acc, g, degp, b, Wh, bh)


def kernel(x, edge_index, W1, b1, W2, b2, Wh, bh):
    N, F = x.shape
    H = W1.shape[1]
    A = Wh.shape[1]
    E = edge_index.shape[1]

    grain = _NS * _CHUNK
    M = -((N + 1) // -grain) * grain
    step_e = _GRP * _CHUNK
    n_chunks = -(E // -(_NW * step_e))
    n_chunks += n_chunks % 2
    e_pad = _NW * step_e * n_chunks - E

    src = jnp.concatenate(
        [edge_index[0], jnp.zeros((e_pad,), jnp.int32)]).reshape(_NW, n_chunks, step_e)
    dst = jnp.concatenate(
        [edge_index[1], jnp.full((e_pad,), N, jnp.int32)]).reshape(_NW, n_chunks, step_e)
    xp = jnp.pad(x, ((0, M - N), (0, 0)))

    ones8 = jnp.ones((step_e, 8), jnp.float32)
    zeros8 = jnp.zeros((_CHUNK, 8), jnp.float32)
    zerosH = jnp.zeros((_CHUNK, H), jnp.float32)

    R = M // 8

    degp = _deg_partials(dst, ones8, zeros8, M, n_chunks)
    g1 = _tc_first(xp, W1, degp, R)
    acc1 = _agg_partials(g1, src, dst, zerosH, M, n_chunks, H)
    g2 = _tc_mid(acc1, g1, degp, b1.reshape(1, H), W2, R)
    acc2 = _agg_partials(g2, src, dst, zerosH, M, n_chunks, H)
    out = _tc_head(acc2, g2, degp, b2.reshape(1, H), Wh, bh.reshape(1, A), R)
    return out[:N]


# speedup vs baseline: 1.0138x; 1.0138x over previous
import functools

import jax
import jax.numpy as jnp
from jax import lax
from jax.experimental import pallas as pl
from jax.experimental.pallas import tpu as pltpu
from jax.experimental.pallas import tpu_sc as plsc

_NC = 2
_NS = 16
_NW = _NC * _NS
_CHUNK = 128
_GRP = 1
_K = 4


def _sc_mesh():
    return plsc.VectorSubcoreMesh(core_axis_name="c", subcore_axis_name="s")


def _deg_partials(dst, ones8, zeros8, M, n_chunks):
    rpt = M // _NS

    def body(dst_hbm, ones_hbm, zeros_hbm, out_hbm, ones_v, zbuf_v, dst_v, acc_sh):
        c = lax.axis_index("c")
        s = lax.axis_index("s")
        wid = c * _NS + s
        pltpu.sync_copy(zeros_hbm, zbuf_v)
        for k in range(rpt // _CHUNK):
            pltpu.sync_copy(zbuf_v, acc_sh.at[pl.ds(s * rpt + k * _CHUNK, _CHUNK)])
        pltpu.sync_copy(dst_hbm.at[wid], dst_v)
        pltpu.sync_copy(ones_hbm, ones_v)
        plsc.subcore_barrier()

        def step(i, carry):
            pltpu.sync_copy(ones_v, acc_sh.at[dst_v.at[i]], add=True)
            return carry

        lax.fori_loop(0, n_chunks, step, 0)
        plsc.subcore_barrier()
        pltpu.sync_copy(acc_sh.at[pl.ds(s * rpt, rpt)],
                        out_hbm.at[c, pl.ds(s * rpt, rpt)])

    fn = pl.kernel(
        body,
        out_type=jax.ShapeDtypeStruct((_NC, M, 8), jnp.float32),
        mesh=_sc_mesh(),
        scratch_types=[
            pltpu.VMEM((_GRP * _CHUNK, 8), jnp.float32),
            pltpu.VMEM((_CHUNK, 8), jnp.float32),
            pltpu.VMEM((n_chunks, _GRP * _CHUNK), jnp.int32),
            pltpu.VMEM_SHARED((M, 8), jnp.float32),
        ],
    )
    return fn(dst, ones8, zeros8)


def _agg_partials(g, src, dst, zeros_h, M, n_chunks, H):
    rpt = M // _NS

    def body(g_hbm, src_hbm, dst_hbm, zeros_hbm, out_hbm,
             src_v, dst_v, rows_v, zbuf_v, acc_sh, semg, sems):
        c = lax.axis_index("c")
        s = lax.axis_index("s")
        wid = c * _NS + s
        pltpu.sync_copy(zeros_hbm, zbuf_v)
        for k in range(rpt // _CHUNK):
            pltpu.sync_copy(zbuf_v, acc_sh.at[pl.ds(s * rpt + k * _CHUNK, _CHUNK)])
        pltpu.sync_copy(src_hbm.at[wid], src_v)
        pltpu.sync_copy(dst_hbm.at[wid], dst_v)
        plsc.subcore_barrier()

        def step(jj, carry):
            j = _K * jj
            gds = [pltpu.async_copy(g_hbm.at[src_v.at[j + b]], rows_v.at[b], semg)
                   for b in range(_K)]
            for d in gds:
                d.wait()
            sds = [pltpu.async_copy(rows_v.at[b], acc_sh.at[dst_v.at[j + b]], sems,
                                    add=True)
                   for b in range(_K)]
            for d in sds:
                d.wait()
            return carry

        lax.fori_loop(0, n_chunks // _K, step, 0)
        plsc.subcore_barrier()
        pltpu.sync_copy(acc_sh.at[pl.ds(s * rpt, rpt)],
                        out_hbm.at[c, pl.ds(s * rpt, rpt)])

    fn = pl.kernel(
        body,
        out_type=jax.ShapeDtypeStruct((_NC, M, H), jnp.float32),
        mesh=_sc_mesh(),
        scratch_types=[
            pltpu.VMEM((n_chunks, _CHUNK), jnp.int32),
            pltpu.VMEM((n_chunks, _CHUNK), jnp.int32),
            pltpu.VMEM((_K, _CHUNK, H), jnp.float32),
            pltpu.VMEM((_CHUNK, H), jnp.float32),
            pltpu.VMEM_SHARED((M, H), jnp.float32),
            pltpu.SemaphoreType.DMA,
            pltpu.SemaphoreType.DMA,
        ],
        compiler_params=pltpu.CompilerParams(use_tc_tiling_on_sc=False),
    )
    return fn(g, src, dst, zeros_h)


def _dinv(dp_ref):
    return lax.rsqrt(dp_ref[0, :, 0:1] + dp_ref[1, :, 0:1] + 1.0)


def _tc_first(xp, W1, degp, R):
    M, F = xp.shape
    H = W1.shape[1]

    def body(x_ref, w_ref, dp_ref, g_ref):
        h = jnp.dot(x_ref[...], w_ref[...], preferred_element_type=jnp.float32)
        g_ref[...] = _dinv(dp_ref) * h

    return pl.pallas_call(
        body,
        grid=(M // R,),
        in_specs=[
            pl.BlockSpec((R, F), lambda i: (i, 0)),
            pl.BlockSpec((F, H), lambda i: (0, 0)),
            pl.BlockSpec((2, R, 8), lambda i: (0, i, 0)),
        ],
        out_specs=pl.BlockSpec((R, H), lambda i: (i, 0)),
        out_shape=jax.ShapeDtypeStruct((M, H), jnp.float32),
    )(xp, W1, degp)


def _tc_mid(acc, g, degp, b, W, R):
    M, H = g.shape
    H2 = W.shape[1]

    def body(a_ref, g_ref, dp_ref, b_ref, w_ref, o_ref):
        d = _dinv(dp_ref)
        t = jnp.maximum(d * (a_ref[0] + a_ref[1] + g_ref[...]) + b_ref[...], 0.0)
        o_ref[...] = d * jnp.dot(t, w_ref[...], preferred_element_type=jnp.float32)

    return pl.pallas_call(
        body,
        grid=(M // R,),
        in_specs=[
            pl.BlockSpec((2, R, H), lambda i: (0, i, 0)),
            pl.BlockSpec((R, H), lambda i: (i, 0)),
            pl.BlockSpec((2, R, 8), lambda i: (0, i, 0)),
            pl.BlockSpec((1, H), lambda i: (0, 0)),
            pl.BlockSpec((H, H2), lambda i: (0, 0)),
        ],
        out_specs=pl.BlockSpec((R, H2), lambda i: (i, 0)),
        out_shape=jax.ShapeDtypeStruct((M, H2), jnp.float32),
    )(acc, g, degp, b, W)


def _tc_head(acc, g, degp, b, Wh, bh, R):
    M, H = g.shape
    A = Wh.shape[1]

    def body(a_ref, g_ref, dp_ref, b_ref, w_ref, bh_ref, o_ref):
        d = _dinv(dp_ref)
        t = jnp.maximum(d * (a_ref[0] + a_ref[1] + g_ref[...]) + b_ref[...], 0.0)
        o_ref[...] = jnp.dot(t, w_ref[...], preferred_element_type=jnp.float32) + bh_ref[...]

    return pl.pallas_call(
        body,
        grid=(M // R,),
        in_specs=[
            pl.BlockSpec((2, R, H), lambda i: (0, i, 0)),
            pl.BlockSpec((R, H), lambda i: (i, 0)),
            pl.BlockSpec((2, R, 8), lambda i: (0, i, 0)),
            pl.BlockSpec((1, H), lambda i: (0, 0)),
            pl.BlockSpec((H, A), lambda i: (0, 0)),
            pl.BlockSpec((1, A), lambda i: (0, 0)),
        ],
        out_specs=pl.BlockSpec((R, A), lambda i: (i, 0)),
        out_shape=jax.ShapeDtypeStruct((M, A), jnp.float32),
    )(acc, g, degp, b, Wh, bh)


def kernel(x, edge_index, W1, b1, W2, b2, Wh, bh):
    N, F = x.shape
    H = W1.shape[1]
    A = Wh.shape[1]
    E = edge_index.shape[1]

    grain = _NS * _CHUNK
    M = -((N + 1) // -grain) * grain
    step_e = _GRP * _CHUNK
    n_chunks = -(E // -(_NW * step_e))
    n_chunks = -(n_chunks // -_K) * _K
    e_pad = _NW * step_e * n_chunks - E

    src = jnp.concatenate(
        [edge_index[0], jnp.zeros((e_pad,), jnp.int32)]).reshape(_NW, n_chunks, step_e)
    dst = jnp.concatenate(
        [edge_index[1], jnp.full((e_pad,), N, jnp.int32)]).reshape(_NW, n_chunks, step_e)
    xp = jnp.pad(x, ((0, M - N), (0, 0)))

    ones8 = jnp.ones((step_e, 8), jnp.float32)
    zeros8 = jnp.zeros((_CHUNK, 8), jnp.float32)
    zerosH = jnp.zeros((_CHUNK, H), jnp.float32)

    R = M // 8

    degp = _deg_partials(dst, ones8, zeros8, M, n_chunks)
    g1 = _tc_first(xp, W1, degp, R)
    acc1 = _agg_partials(g1, src, dst, zerosH, M, n_chunks, H)
    g2 = _tc_mid(acc1, g1, degp, b1.reshape(1, H), W2, R)
    acc2 = _agg_partials(g2, src, dst, zerosH, M, n_chunks, H)
    out = _tc_head(acc2, g2, degp, b2.reshape(1, H), Wh, bh.reshape(1, A), R)
    return out[:N]


# speedup vs baseline: 1.8992x; 1.8733x over previous
import functools

import jax
import jax.numpy as jnp
from jax import lax
from jax.experimental import pallas as pl
from jax.experimental.pallas import tpu as pltpu
from jax.experimental.pallas import tpu_sc as plsc

_NC = 2
_NS = 16
_NW = _NC * _NS
_CHUNK = 128
_GRP = 1


def _sc_mesh():
    return plsc.VectorSubcoreMesh(core_axis_name="c", subcore_axis_name="s")


def _deg_partials(dst, ones8, zeros8, M, n_chunks):
    rpt = M // _NS

    def body(dst_hbm, ones_hbm, zeros_hbm, out_hbm, ones_v, zbuf_v, dst_v, acc_sh):
        c = lax.axis_index("c")
        s = lax.axis_index("s")
        wid = c * _NS + s
        pltpu.sync_copy(zeros_hbm, zbuf_v)
        for k in range(rpt // _CHUNK):
            pltpu.sync_copy(zbuf_v, acc_sh.at[pl.ds(s * rpt + k * _CHUNK, _CHUNK)])
        pltpu.sync_copy(dst_hbm.at[wid], dst_v)
        pltpu.sync_copy(ones_hbm, ones_v)
        plsc.subcore_barrier()

        def step(i, carry):
            pltpu.sync_copy(ones_v, acc_sh.at[dst_v.at[i]], add=True)
            return carry

        lax.fori_loop(0, n_chunks, step, 0)
        plsc.subcore_barrier()
        pltpu.sync_copy(acc_sh.at[pl.ds(s * rpt, rpt)],
                        out_hbm.at[c, pl.ds(s * rpt, rpt)])

    fn = pl.kernel(
        body,
        out_type=jax.ShapeDtypeStruct((_NC, M, 8), jnp.float32),
        mesh=_sc_mesh(),
        scratch_types=[
            pltpu.VMEM((_GRP * _CHUNK, 8), jnp.float32),
            pltpu.VMEM((_CHUNK, 8), jnp.float32),
            pltpu.VMEM((n_chunks, _GRP * _CHUNK), jnp.int32),
            pltpu.VMEM_SHARED((M, 8), jnp.float32),
        ],
    )
    return fn(dst, ones8, zeros8)


def _agg_partials(g, src, dst, zeros_h, M, n_chunks, H):
    rpt = M // _NS

    def body(g_hbm, src_hbm, dst_hbm, zeros_hbm, out_hbm,
             src_v, dst_v, rows_v, zbuf_v, g_sh, acc_sh, semg):
        c = lax.axis_index("c")
        s = lax.axis_index("s")
        wid = c * _NS + s
        pltpu.sync_copy(zeros_hbm, zbuf_v)
        for k in range(rpt // _CHUNK):
            pltpu.sync_copy(zbuf_v, acc_sh.at[pl.ds(s * rpt + k * _CHUNK, _CHUNK)])
        pltpu.sync_copy(g_hbm.at[pl.ds(s * rpt, rpt)], g_sh.at[pl.ds(s * rpt, rpt)])
        pltpu.sync_copy(src_hbm.at[wid], src_v)
        pltpu.sync_copy(dst_hbm.at[wid], dst_v)
        plsc.subcore_barrier()

        def step(j, carry):
            pltpu.async_copy(g_sh.at[src_v.at[j]], rows_v, semg).wait()
            pltpu.sync_copy(rows_v, acc_sh.at[dst_v.at[j]], add=True)
            return carry

        lax.fori_loop(0, n_chunks, step, 0)
        plsc.subcore_barrier()
        pltpu.sync_copy(acc_sh.at[pl.ds(s * rpt, rpt)],
                        out_hbm.at[c, pl.ds(s * rpt, rpt)])

    fn = pl.kernel(
        body,
        out_type=jax.ShapeDtypeStruct((_NC, M, H), jnp.float32),
        mesh=_sc_mesh(),
        scratch_types=[
            pltpu.VMEM((n_chunks, _CHUNK), jnp.int32),
            pltpu.VMEM((n_chunks, _CHUNK), jnp.int32),
            pltpu.VMEM((_CHUNK, H), jnp.float32),
            pltpu.VMEM((_CHUNK, H), jnp.float32),
            pltpu.VMEM_SHARED((M, H), jnp.float32),
            pltpu.VMEM_SHARED((M, H), jnp.float32),
            pltpu.SemaphoreType.DMA,
        ],
        compiler_params=pltpu.CompilerParams(use_tc_tiling_on_sc=False),
    )
    return fn(g, src, dst, zeros_h)


def _dinv(dp_ref):
    return lax.rsqrt(dp_ref[0, :, 0:1] + dp_ref[1, :, 0:1] + 1.0)


def _tc_first(xp, W1, degp, R):
    M, F = xp.shape
    H = W1.shape[1]

    def body(x_ref, w_ref, dp_ref, g_ref):
        h = jnp.dot(x_ref[...], w_ref[...], preferred_element_type=jnp.float32)
        g_ref[...] = _dinv(dp_ref) * h

    return pl.pallas_call(
        body,
        grid=(M // R,),
        in_specs=[
            pl.BlockSpec((R, F), lambda i: (i, 0)),
            pl.BlockSpec((F, H), lambda i: (0, 0)),
            pl.BlockSpec((2, R, 8), lambda i: (0, i, 0)),
        ],
        out_specs=pl.BlockSpec((R, H), lambda i: (i, 0)),
        out_shape=jax.ShapeDtypeStruct((M, H), jnp.float32),
    )(xp, W1, degp)


def _tc_mid(acc, g, degp, b, W, R):
    M, H = g.shape
    H2 = W.shape[1]

    def body(a_ref, g_ref, dp_ref, b_ref, w_ref, o_ref):
        d = _dinv(dp_ref)
        t = jnp.maximum(d * (a_ref[0] + a_ref[1] + g_ref[...]) + b_ref[...], 0.0)
        o_ref[...] = d * jnp.dot(t, w_ref[...], preferred_element_type=jnp.float32)

    return pl.pallas_call(
        body,
        grid=(M // R,),
        in_specs=[
            pl.BlockSpec((2, R, H), lambda i: (0, i, 0)),
            pl.BlockSpec((R, H), lambda i: (i, 0)),
            pl.BlockSpec((2, R, 8), lambda i: (0, i, 0)),
            pl.BlockSpec((1, H), lambda i: (0, 0)),
            pl.BlockSpec((H, H2), lambda i: (0, 0)),
        ],
        out_specs=pl.BlockSpec((R, H2), lambda i: (i, 0)),
        out_shape=jax.ShapeDtypeStruct((M, H2), jnp.float32),
    )(acc, g, degp, b, W)


def _tc_head(acc, g, degp, b, Wh, bh, R):
    M, H = g.shape
    A = Wh.shape[1]

    def body(a_ref, g_ref, dp_ref, b_ref, w_ref, bh_ref, o_ref):
        d = _dinv(dp_ref)
        t = jnp.maximum(d * (a_ref[0] + a_ref[1] + g_ref[...]) + b_ref[...], 0.0)
        o_ref[...] = jnp.dot(t, w_ref[...], preferred_element_type=jnp.float32) + bh_ref[...]

    return pl.pallas_call(
        body,
        grid=(M // R,),
        in_specs=[
            pl.BlockSpec((2, R, H), lambda i: (0, i, 0)),
            pl.BlockSpec((R, H), lambda i: (i, 0)),
            pl.BlockSpec((2, R, 8), lambda i: (0, i, 0)),
            pl.BlockSpec((1, H), lambda i: (0, 0)),
            pl.BlockSpec((H, A), lambda i: (0, 0)),
            pl.BlockSpec((1, A), lambda i: (0, 0)),
        ],
        out_specs=pl.BlockSpec((R, A), lambda i: (i, 0)),
        out_shape=jax.ShapeDtypeStruct((M, A), jnp.float32),
    )(acc, g, degp, b, Wh, bh)


def kernel(x, edge_index, W1, b1, W2, b2, Wh, bh):
    N, F = x.shape
    H = W1.shape[1]
    A = Wh.shape[1]
    E = edge_index.shape[1]

    grain = _NS * _CHUNK
    M = -((N + 1) // -grain) * grain
    step_e = _GRP * _CHUNK
    n_chunks = -(E // -(_NW * step_e))

    e_pad = _NW * step_e * n_chunks - E

    src = jnp.concatenate(
        [edge_index[0], jnp.zeros((e_pad,), jnp.int32)]).reshape(_NW, n_chunks, step_e)
    dst = jnp.concatenate(
        [edge_index[1], jnp.full((e_pad,), N, jnp.int32)]).reshape(_NW, n_chunks, step_e)
    xp = jnp.pad(x, ((0, M - N), (0, 0)))

    ones8 = jnp.ones((step_e, 8), jnp.float32)
    zeros8 = jnp.zeros((_CHUNK, 8), jnp.float32)
    zerosH = jnp.zeros((_CHUNK, H), jnp.float32)

    R = M // 8

    degp = _deg_partials(dst, ones8, zeros8, M, n_chunks)
    g1 = _tc_first(xp, W1, degp, R)
    acc1 = _agg_partials(g1, src, dst, zerosH, M, n_chunks, H)
    g2 = _tc_mid(acc1, g1, degp, b1.reshape(1, H), W2, R)
    acc2 = _agg_partials(g2, src, dst, zerosH, M, n_chunks, H)
    out = _tc_head(acc2, g2, degp, b2.reshape(1, H), Wh, bh.reshape(1, A), R)
    return out[:N]
